# Initial kernel scaffold; baseline (speedup 1.0000x reference)
import jax
import jax.numpy as jnp
from jax.experimental import pallas as pl


def kernel(weekdays, start_mins, durations, time_diffs, weekday_table, hour_table, time_diff_table, duration_table, duration_bins, W, b, gamma, beta):
    raise NotImplementedError("write your pallas kernel here")


# speedup vs baseline: 25.1615x; 25.1615x over previous
import math

import jax
import jax.numpy as jnp
from jax import lax
from jax.experimental import pallas as pl

_B, _L, _H = 4096, 200, 64
_NT = _B * _L
_BL = 2048
_ROWS = _NT // _BL
_RPB = 8
_GRID = _ROWS // _RPB
_TPB = _RPB * _BL


def _fuse_body(e_ref, w_ref, b_ref, m_ref):
    m0 = jnp.dot(e_ref[...], w_ref[...], preferred_element_type=jnp.float32)
    sel = (lax.broadcasted_iota(jnp.int32, (64, 64), 0) == 51).astype(jnp.float32)
    m0 = m0 + sel * b_ref[...]
    m_ref[...] = m0 - jnp.mean(m0, axis=1, keepdims=True)


def _main_body(wd_ref, sm_ref, du_ref, td_ref, m_ref, bins_ref, g_ref, be_ref, o_ref):
    m = m_ref[...]
    g = g_ref[...]
    be = be_ref[...]
    bins = bins_ref[...]
    for r in range(_RPB):
        wd = wd_ref[r:r + 1, :]
        sm = sm_ref[r:r + 1, :]
        du = du_ref[r:r + 1, :]
        td = td_ref[r:r + 1, :]
        hr = jnp.clip(sm // 60, 0, 23)
        theta = sm.astype(jnp.float32) * jnp.float32(2.0 * math.pi / 1440.0)
        sinv = jnp.sin(theta)
        cosv = jnp.cos(theta)
        ld = jnp.log1p(du)
        cnt = jnp.sum((bins < ld).astype(jnp.int32), axis=0, keepdims=True)
        db = jnp.clip(cnt - 1, 0, 9)
        k = lax.broadcasted_iota(jnp.int32, (64, _BL), 0)
        oh_int = ((k == wd) | (k == hr + 7) | (k == db + 31) | (k == td + 41)
                  | (k == 51))
        oh = oh_int.astype(jnp.float32)
        oh = oh + jnp.where(k == 49, sinv, 0.0) + jnp.where(k == 50, cosv, 0.0)
        h = lax.dot_general(oh, m, (((0,), (0,)), ((), ())),
                            preferred_element_type=jnp.float32)
        var = jnp.mean(h * h, axis=1, keepdims=True)
        inv = lax.rsqrt(var + 1e-5)
        o_ref[r * _BL:(r + 1) * _BL, :] = h * inv * g + be


def kernel(weekdays, start_mins, durations, time_diffs, weekday_table,
           hour_table, time_diff_table, duration_table, duration_bins,
           W, b, gamma, beta):
    f32 = jnp.float32
    wd2 = weekdays.astype(jnp.int32).reshape(_ROWS, _BL)
    sm2 = start_mins.astype(jnp.int32).reshape(_ROWS, _BL)
    du2 = durations.astype(f32).reshape(_ROWS, _BL)
    td2 = time_diffs.astype(jnp.int32).reshape(_ROWS, _BL)

    E = jnp.zeros((64, 48), f32)
    E = E.at[0:7, 0:12].set(weekday_table.astype(f32))
    E = E.at[7:31, 12:24].set(hour_table.astype(f32))
    E = E.at[31:41, 26:34].set(duration_table.astype(f32))
    E = E.at[41:49, 34:42].set(time_diff_table.astype(f32))
    E = E.at[49, 24].set(1.0)
    E = E.at[50, 25].set(1.0)
    Wp = jnp.zeros((48, 64), f32).at[0:42, :].set(W.astype(f32))

    Mc = pl.pallas_call(
        _fuse_body,
        out_shape=jax.ShapeDtypeStruct((64, 64), f32),
    )(E, Wp, b.astype(f32).reshape(1, 64))

    bins_col = jnp.full((16, 1), jnp.inf, f32).at[0:10, 0].set(
        duration_bins.astype(f32))

    out2 = pl.pallas_call(
        _main_body,
        grid=(_GRID,),
        in_specs=[
            pl.BlockSpec((_RPB, _BL), lambda i: (i, 0)),
            pl.BlockSpec((_RPB, _BL), lambda i: (i, 0)),
            pl.BlockSpec((_RPB, _BL), lambda i: (i, 0)),
            pl.BlockSpec((_RPB, _BL), lambda i: (i, 0)),
            pl.BlockSpec((64, 64), lambda i: (0, 0)),
            pl.BlockSpec((16, 1), lambda i: (0, 0)),
            pl.BlockSpec((1, 64), lambda i: (0, 0)),
            pl.BlockSpec((1, 64), lambda i: (0, 0)),
        ],
        out_specs=pl.BlockSpec((_TPB, 64), lambda i: (i, 0)),
        out_shape=jax.ShapeDtypeStruct((_NT, 64), f32),
    )(wd2, sm2, du2, td2, Mc, bins_col,
      gamma.astype(f32).reshape(1, 64), beta.astype(f32).reshape(1, 64))

    return out2.reshape(_B, _L, _H)


# speedup vs baseline: 27.2498x; 1.0830x over previous
import math

import jax
import jax.numpy as jnp
from jax import lax
from jax.experimental import pallas as pl

_B, _L, _H = 4096, 200, 64
_NT = _B * _L
_BL = 2048
_ROWS = _NT // _BL
_RPB = 8
_GRID = _ROWS // _RPB
_TPB = _RPB * _BL


def _fuse_body(e_ref, w_ref, b_ref, m_ref):
    m0 = jnp.dot(e_ref[...], w_ref[...], preferred_element_type=jnp.float32)
    sel = (lax.broadcasted_iota(jnp.int32, (64, 64), 0) == 51).astype(jnp.float32)
    m0 = m0 + sel * b_ref[...]
    m_ref[...] = m0 - jnp.mean(m0, axis=1, keepdims=True)


def _main_body(wd_ref, sm_ref, du_ref, td_ref, m_ref, bins_ref, g_ref, be_ref, o_ref):
    m = m_ref[...].astype(jnp.bfloat16)
    g = g_ref[...]
    be = be_ref[...]
    bins = bins_ref[...]
    ones = jnp.full((64, 64), 1.0 / 64.0, jnp.bfloat16)
    k = lax.broadcasted_iota(jnp.int16, (64, _BL), 0).astype(jnp.bfloat16)
    for r in range(_RPB):
        wd = wd_ref[r:r + 1, :]
        sm = sm_ref[r:r + 1, :]
        du = du_ref[r:r + 1, :]
        td = td_ref[r:r + 1, :]
        hr = jnp.clip(sm // 60, 0, 23)
        theta = sm.astype(jnp.float32) * jnp.float32(2.0 * math.pi / 1440.0)
        sinv = jnp.sin(theta).astype(jnp.bfloat16)
        cosv = jnp.cos(theta).astype(jnp.bfloat16)
        ld = jnp.log1p(du)
        cnt = jnp.sum((bins < ld).astype(jnp.int32), axis=0, keepdims=True)
        db = jnp.clip(cnt - 1, 0, 9)
        bf = jnp.bfloat16
        wd_b = wd.astype(bf)
        hr_b = (hr + 7).astype(bf)
        db_b = (db + 31).astype(bf)
        td_b = (td + 41).astype(bf)
        hit = ((k == wd_b) | (k == hr_b) | (k == db_b) | (k == td_b)
               | (k == bf(51.0)))
        oh = jnp.where(hit, bf(1.0),
                       jnp.where(k == bf(49.0), sinv,
                                 jnp.where(k == bf(50.0), cosv, bf(0.0))))
        h = lax.dot_general(oh, m, (((0,), (0,)), ((), ())),
                            preferred_element_type=jnp.float32)
        hh = (h * h).astype(jnp.bfloat16)
        var = jnp.dot(hh, ones, preferred_element_type=jnp.float32)
        inv = lax.rsqrt(var + 1e-5)
        o_ref[r * _BL:(r + 1) * _BL, :] = h * inv * g + be


def kernel(weekdays, start_mins, durations, time_diffs, weekday_table,
           hour_table, time_diff_table, duration_table, duration_bins,
           W, b, gamma, beta):
    f32 = jnp.float32
    wd2 = weekdays.astype(jnp.int32).reshape(_ROWS, _BL)
    sm2 = start_mins.astype(jnp.int32).reshape(_ROWS, _BL)
    du2 = durations.astype(f32).reshape(_ROWS, _BL)
    td2 = time_diffs.astype(jnp.int32).reshape(_ROWS, _BL)

    E = jnp.zeros((64, 48), f32)
    E = E.at[0:7, 0:12].set(weekday_table.astype(f32))
    E = E.at[7:31, 12:24].set(hour_table.astype(f32))
    E = E.at[31:41, 26:34].set(duration_table.astype(f32))
    E = E.at[41:49, 34:42].set(time_diff_table.astype(f32))
    E = E.at[49, 24].set(1.0)
    E = E.at[50, 25].set(1.0)
    Wp = jnp.zeros((48, 64), f32).at[0:42, :].set(W.astype(f32))

    Mc = pl.pallas_call(
        _fuse_body,
        out_shape=jax.ShapeDtypeStruct((64, 64), f32),
    )(E, Wp, b.astype(f32).reshape(1, 64))

    bins_col = jnp.full((16, 1), jnp.inf, f32).at[0:10, 0].set(
        duration_bins.astype(f32))

    out2 = pl.pallas_call(
        _main_body,
        grid=(_GRID,),
        in_specs=[
            pl.BlockSpec((_RPB, _BL), lambda i: (i, 0)),
            pl.BlockSpec((_RPB, _BL), lambda i: (i, 0)),
            pl.BlockSpec((_RPB, _BL), lambda i: (i, 0)),
            pl.BlockSpec((_RPB, _BL), lambda i: (i, 0)),
            pl.BlockSpec((64, 64), lambda i: (0, 0)),
            pl.BlockSpec((16, 1), lambda i: (0, 0)),
            pl.BlockSpec((1, 64), lambda i: (0, 0)),
            pl.BlockSpec((1, 64), lambda i: (0, 0)),
        ],
        out_specs=pl.BlockSpec((_TPB, 64), lambda i: (i, 0)),
        out_shape=jax.ShapeDtypeStruct((_NT, 64), f32),
    )(wd2, sm2, du2, td2, Mc, bins_col,
      gamma.astype(f32).reshape(1, 64), beta.astype(f32).reshape(1, 64))

    return out2.reshape(_B, _L, _H)
